# baseline (speedup 1.0000x reference)
import jax
import jax.numpy as jnp
from jax.experimental import pallas as pl
from jax.experimental.pallas import tpu as pltpu


def _cdot(wt, h3):
    return jax.lax.dot_general(wt, h3, (((0,), (0,)), ((), ())),
                               preferred_element_type=jnp.float32)


def _cdot_r(w, h3):
    return jax.lax.dot_general(w, h3, (((1,), (0,)), ((), ())),
                               preferred_element_type=jnp.float32)


def _fused_body(x_ref, a_ref, w1_ref, b1_ref, w2_ref, b2_ref,
                w3_ref, b3_ref, wc_ref, bc_ref, out_ref, aout_ref):
    f32 = jnp.float32
    c = out_ref.shape[1]
    k = a_ref.shape[1]

    h = a_ref[0, :k - 1]
    h = jnp.maximum(_cdot(w1_ref[...], h) + b1_ref[...][:, :, None], 0.0)
    h = jnp.maximum(_cdot(w2_ref[...], h) + b2_ref[...][:, :, None], 0.0)
    h = jnp.maximum(_cdot(w3_ref[...], h) + b3_ref[...][:, :, None], 0.0)
    aout3 = h * a_ref[0, k - 1:]
    aout_ref[0] = aout3

    xc3 = _cdot_r(wc_ref[...], x_ref[0]) + bc_ref[...][:, :, None]
    xc3h = xc3.astype(jnp.bfloat16)
    aout3h = aout3.astype(jnp.bfloat16)
    for j in range(c):
        out_ref[0, j] = jax.lax.dot_general(
            aout3h[j], xc3h[j], (((0,), (0,)), ((), ())),
            preferred_element_type=f32)


def kernel(x, A, W1, b1, W2, b2, W3, b3, Wc, bc):
    n, c, t, v = x.shape
    k = A.shape[1]

    xt = jnp.swapaxes(x, 2, 3)
    w1t = jnp.swapaxes(W1, 0, 1)
    w2t = jnp.swapaxes(W2, 0, 1)
    w3t = jnp.swapaxes(W3, 0, 1)
    b1c = b1.reshape(-1, 1)
    b2c = b2.reshape(-1, 1)
    b3c = b3.reshape(-1, 1)
    bcc = bc.reshape(-1, 1)

    full = lambda a: pl.BlockSpec(a.shape, lambda i: (0,) * a.ndim)

    outt, aout = pl.pallas_call(
        _fused_body,
        grid=(n,),
        compiler_params=pltpu.CompilerParams(
            dimension_semantics=("parallel",)),
        in_specs=[
            pl.BlockSpec((1, c, v, t), lambda i: (i, 0, 0, 0)),
            pl.BlockSpec((1, k, v, v), lambda i: (i, 0, 0, 0)),
            full(w1t), full(b1c), full(w2t), full(b2c),
            full(w3t), full(b3c), full(Wc), full(bcc),
        ],
        out_specs=[
            pl.BlockSpec((1, c, v, t), lambda i: (i, 0, 0, 0)),
            pl.BlockSpec((1, c, v, v), lambda i: (i, 0, 0, 0)),
        ],
        out_shape=[
            jax.ShapeDtypeStruct((n, c, v, t), jnp.float32),
            jax.ShapeDtypeStruct((n, c, v, v), jnp.float32),
        ],
    )(xt, A, w1t, b1c, w2t, b2c, w3t, b3c, Wc, bcc)

    out = jnp.swapaxes(outt, 2, 3)
    return (out, aout)


# speedup vs baseline: 1.0001x; 1.0001x over previous
import jax
import jax.numpy as jnp
from jax.experimental import pallas as pl
from jax.experimental.pallas import tpu as pltpu


def _cdot(wt, h3):
    return jax.lax.dot_general(wt, h3, (((0,), (0,)), ((), ())),
                               preferred_element_type=jnp.float32)


def _cdot_r(w, h3):
    return jax.lax.dot_general(w, h3, (((1,), (0,)), ((), ())),
                               preferred_element_type=jnp.float32)


def _fused_body(x_ref, a_ref, w1_ref, b1_ref, w2_ref, b2_ref,
                w3_ref, b3_ref, wc_ref, bc_ref, out_ref, aout_ref):
    f32 = jnp.float32
    c = out_ref.shape[1]
    k = a_ref.shape[1]

    w18 = jnp.pad(w1_ref[...], ((0, 1), (0, 0)))
    h = jnp.maximum(_cdot(w18, a_ref[0]) + b1_ref[...][:, :, None], 0.0)
    h = jnp.maximum(_cdot(w2_ref[...], h) + b2_ref[...][:, :, None], 0.0)
    h = jnp.maximum(_cdot(w3_ref[...], h) + b3_ref[...][:, :, None], 0.0)
    aout3 = h * a_ref[0, k - 1:]
    aout_ref[0] = aout3

    xc3 = _cdot_r(wc_ref[...], x_ref[0]) + bc_ref[...][:, :, None]
    xc3h = xc3.astype(jnp.bfloat16)
    aout3h = aout3.astype(jnp.bfloat16)
    for j in range(c):
        out_ref[0, j] = jax.lax.dot_general(
            aout3h[j], xc3h[j], (((0,), (0,)), ((), ())),
            preferred_element_type=f32)


def kernel(x, A, W1, b1, W2, b2, W3, b3, Wc, bc):
    n, c, t, v = x.shape
    k = A.shape[1]

    xt = jnp.swapaxes(x, 2, 3)
    w1t = jnp.swapaxes(W1, 0, 1)
    w2t = jnp.swapaxes(W2, 0, 1)
    w3t = jnp.swapaxes(W3, 0, 1)
    b1c = b1.reshape(-1, 1)
    b2c = b2.reshape(-1, 1)
    b3c = b3.reshape(-1, 1)
    bcc = bc.reshape(-1, 1)

    full = lambda a: pl.BlockSpec(a.shape, lambda i: (0,) * a.ndim)

    outt, aout = pl.pallas_call(
        _fused_body,
        grid=(n,),
        compiler_params=pltpu.CompilerParams(
            dimension_semantics=("parallel",)),
        in_specs=[
            pl.BlockSpec((1, c, v, t), lambda i: (i, 0, 0, 0)),
            pl.BlockSpec((1, k, v, v), lambda i: (i, 0, 0, 0)),
            full(w1t), full(b1c), full(w2t), full(b2c),
            full(w3t), full(b3c), full(Wc), full(bcc),
        ],
        out_specs=[
            pl.BlockSpec((1, c, v, t), lambda i: (i, 0, 0, 0)),
            pl.BlockSpec((1, c, v, v), lambda i: (i, 0, 0, 0)),
        ],
        out_shape=[
            jax.ShapeDtypeStruct((n, c, v, t), jnp.float32),
            jax.ShapeDtypeStruct((n, c, v, v), jnp.float32),
        ],
    )(xt, A, w1t, b1c, w2t, b2c, w3t, b3c, Wc, bcc)

    out = jnp.swapaxes(outt, 2, 3)
    return (out, aout)


# speedup vs baseline: 1.1048x; 1.1047x over previous
import jax
import jax.numpy as jnp
from jax.experimental import pallas as pl
from jax.experimental.pallas import tpu as pltpu


def _cdot(wt, h3):
    return jax.lax.dot_general(wt, h3, (((0,), (0,)), ((), ())),
                               preferred_element_type=jnp.float32)


def _cdot_r(w, h3):
    return jax.lax.dot_general(w, h3, (((1,), (0,)), ((), ())),
                               preferred_element_type=jnp.float32)


def _fused_body(x_ref, a_ref, w1_ref, b1_ref, w2_ref, b2_ref,
                w3_ref, b3_ref, wc_ref, bc_ref, out_ref, aout_ref):
    f32 = jnp.float32
    c = out_ref.shape[1]
    k = a_ref.shape[1]

    _b = lambda r: r[...][:, None, None]
    w18 = jnp.pad(w1_ref[...], ((0, 1), (0, 0)))
    h = jnp.maximum(_cdot(w18, a_ref[0]) + _b(b1_ref), 0.0)
    h = jnp.maximum(_cdot(w2_ref[...], h) + _b(b2_ref), 0.0)
    h = jnp.maximum(_cdot(w3_ref[...], h) + _b(b3_ref), 0.0)
    aout3 = h * a_ref[0, k - 1:]
    aout_ref[0] = aout3

    xc3 = _cdot_r(wc_ref[...], x_ref[0]) + _b(bc_ref)
    xc3h = xc3.astype(jnp.bfloat16)
    aout3h = aout3.astype(jnp.bfloat16)
    for j in range(c):
        out_ref[0, j] = jax.lax.dot_general(
            aout3h[j], xc3h[j], (((0,), (0,)), ((), ())),
            preferred_element_type=f32)


def kernel(x, A, W1, b1, W2, b2, W3, b3, Wc, bc):
    n, c, t, v = x.shape
    k = A.shape[1]

    xt = jnp.swapaxes(x, 2, 3)
    w1t = jnp.swapaxes(W1, 0, 1)
    w2t = jnp.swapaxes(W2, 0, 1)
    w3t = jnp.swapaxes(W3, 0, 1)
    full = lambda a: pl.BlockSpec(a.shape, lambda i: (0,) * a.ndim)

    outt, aout = pl.pallas_call(
        _fused_body,
        grid=(n,),
        compiler_params=pltpu.CompilerParams(
            dimension_semantics=("parallel",)),
        in_specs=[
            pl.BlockSpec((1, c, v, t), lambda i: (i, 0, 0, 0)),
            pl.BlockSpec((1, k, v, v), lambda i: (i, 0, 0, 0)),
            full(w1t), full(b1), full(w2t), full(b2),
            full(w3t), full(b3), full(Wc), full(bc),
        ],
        out_specs=[
            pl.BlockSpec((1, c, v, t), lambda i: (i, 0, 0, 0)),
            pl.BlockSpec((1, c, v, v), lambda i: (i, 0, 0, 0)),
        ],
        out_shape=[
            jax.ShapeDtypeStruct((n, c, v, t), jnp.float32),
            jax.ShapeDtypeStruct((n, c, v, v), jnp.float32),
        ],
    )(xt, A, w1t, b1, w2t, b2, w3t, b3, Wc, bc)

    out = jnp.swapaxes(outt, 2, 3)
    return (out, aout)


# speedup vs baseline: 1.1527x; 1.0434x over previous
import jax
import jax.numpy as jnp
from jax.experimental import pallas as pl
from jax.experimental.pallas import tpu as pltpu


def _cdot(wt, h3):
    return jax.lax.dot_general(wt, h3, (((0,), (0,)), ((), ())),
                               preferred_element_type=jnp.float32)


def _cdot_r(w, h3):
    return jax.lax.dot_general(w, h3, (((1,), (0,)), ((), ())),
                               preferred_element_type=jnp.float32)


def _fused_body(x_ref, a_ref, w1_ref, b1_ref, w2_ref, b2_ref,
                w3_ref, b3_ref, wc_ref, bc_ref, out_ref, aout_ref):
    f32 = jnp.float32
    c = out_ref.shape[1]
    k = a_ref.shape[1]

    _b = lambda r: r[...][:, None, None]
    w18 = jnp.pad(w1_ref[...], ((0, 1), (0, 0)))
    h = jnp.maximum(_cdot(w18, a_ref[0]) + _b(b1_ref), 0.0)
    h = jnp.maximum(_cdot(w2_ref[...], h) + _b(b2_ref), 0.0)
    h = jnp.maximum(_cdot(w3_ref[...], h) + _b(b3_ref), 0.0)
    aout3 = h * a_ref[0, k - 1:]
    aout_ref[0] = aout3

    xc3 = (_cdot_r(wc_ref[...].astype(jnp.bfloat16),
                   x_ref[0].astype(jnp.bfloat16))
           + _b(bc_ref))
    xc3h = xc3.astype(jnp.bfloat16)
    aout3h = aout3.astype(jnp.bfloat16)
    for j in range(c):
        out_ref[0, j] = jax.lax.dot_general(
            aout3h[j], xc3h[j], (((0,), (0,)), ((), ())),
            preferred_element_type=f32)


def kernel(x, A, W1, b1, W2, b2, W3, b3, Wc, bc):
    n, c, t, v = x.shape
    k = A.shape[1]

    xt = jnp.swapaxes(x, 2, 3)
    w1t = jnp.swapaxes(W1, 0, 1)
    w2t = jnp.swapaxes(W2, 0, 1)
    w3t = jnp.swapaxes(W3, 0, 1)
    full = lambda a: pl.BlockSpec(a.shape, lambda i: (0,) * a.ndim)

    outt, aout = pl.pallas_call(
        _fused_body,
        grid=(n,),
        compiler_params=pltpu.CompilerParams(
            dimension_semantics=("parallel",)),
        in_specs=[
            pl.BlockSpec((1, c, v, t), lambda i: (i, 0, 0, 0)),
            pl.BlockSpec((1, k, v, v), lambda i: (i, 0, 0, 0)),
            full(w1t), full(b1), full(w2t), full(b2),
            full(w3t), full(b3), full(Wc), full(bc),
        ],
        out_specs=[
            pl.BlockSpec((1, c, v, t), lambda i: (i, 0, 0, 0)),
            pl.BlockSpec((1, c, v, v), lambda i: (i, 0, 0, 0)),
        ],
        out_shape=[
            jax.ShapeDtypeStruct((n, c, v, t), jnp.float32),
            jax.ShapeDtypeStruct((n, c, v, v), jnp.float32),
        ],
    )(xt, A, w1t, b1, w2t, b2, w3t, b3, Wc, bc)

    out = jnp.swapaxes(outt, 2, 3)
    return (out, aout)


# speedup vs baseline: 1.1777x; 1.0216x over previous
import jax
import jax.numpy as jnp
from jax.experimental import pallas as pl
from jax.experimental.pallas import tpu as pltpu


def _cdot(wt, h3):
    return jax.lax.dot_general(wt, h3, (((0,), (0,)), ((), ())),
                               preferred_element_type=jnp.float32)


def _cdot_r(w, h3):
    return jax.lax.dot_general(w, h3, (((1,), (0,)), ((), ())),
                               preferred_element_type=jnp.float32)


def _fused_body(x_ref, a_ref, w1_ref, b1_ref, w2_ref, b2_ref,
                w3_ref, b3_ref, wc_ref, bc_ref, out_ref, aout_ref):
    f32 = jnp.float32
    c = out_ref.shape[1]
    k = a_ref.shape[1]

    bf16 = jnp.bfloat16
    _b = lambda r: r[...][:, None, None]
    w18 = jnp.pad(w1_ref[...], ((0, 1), (0, 0)))
    h = jnp.maximum(_cdot(w18.astype(bf16), a_ref[0].astype(bf16))
                    + _b(b1_ref), 0.0)
    h = jnp.maximum(_cdot(w2_ref[...].astype(bf16), h.astype(bf16))
                    + _b(b2_ref), 0.0)
    h = jnp.maximum(_cdot(w3_ref[...].astype(bf16), h.astype(bf16))
                    + _b(b3_ref), 0.0)
    aout3 = h * a_ref[0, k - 1:]
    aout_ref[0] = aout3

    xc3 = (_cdot_r(wc_ref[...].astype(jnp.bfloat16),
                   x_ref[0].astype(jnp.bfloat16))
           + _b(bc_ref))
    xc3h = xc3.astype(jnp.bfloat16)
    aout3h = aout3.astype(jnp.bfloat16)
    for j in range(c):
        out_ref[0, j] = jax.lax.dot_general(
            aout3h[j], xc3h[j], (((0,), (0,)), ((), ())),
            preferred_element_type=f32)


def kernel(x, A, W1, b1, W2, b2, W3, b3, Wc, bc):
    n, c, t, v = x.shape
    k = A.shape[1]

    xt = jnp.swapaxes(x, 2, 3)
    w1t = jnp.swapaxes(W1, 0, 1)
    w2t = jnp.swapaxes(W2, 0, 1)
    w3t = jnp.swapaxes(W3, 0, 1)
    full = lambda a: pl.BlockSpec(a.shape, lambda i: (0,) * a.ndim)

    outt, aout = pl.pallas_call(
        _fused_body,
        grid=(n,),
        compiler_params=pltpu.CompilerParams(
            dimension_semantics=("parallel",)),
        in_specs=[
            pl.BlockSpec((1, c, v, t), lambda i: (i, 0, 0, 0)),
            pl.BlockSpec((1, k, v, v), lambda i: (i, 0, 0, 0)),
            full(w1t), full(b1), full(w2t), full(b2),
            full(w3t), full(b3), full(Wc), full(bc),
        ],
        out_specs=[
            pl.BlockSpec((1, c, v, t), lambda i: (i, 0, 0, 0)),
            pl.BlockSpec((1, c, v, v), lambda i: (i, 0, 0, 0)),
        ],
        out_shape=[
            jax.ShapeDtypeStruct((n, c, v, t), jnp.float32),
            jax.ShapeDtypeStruct((n, c, v, v), jnp.float32),
        ],
    )(xt, A, w1t, b1, w2t, b2, w3t, b3, Wc, bc)

    out = jnp.swapaxes(outt, 2, 3)
    return (out, aout)


# speedup vs baseline: 1.1948x; 1.0146x over previous
import jax
import jax.numpy as jnp
from jax.experimental import pallas as pl
from jax.experimental.pallas import tpu as pltpu


def _cdot(wt, h3):
    return jax.lax.dot_general(wt, h3, (((0,), (0,)), ((), ())),
                               preferred_element_type=jnp.float32)


def _cdot_r(w, h3):
    return jax.lax.dot_general(w, h3, (((1,), (0,)), ((), ())),
                               preferred_element_type=jnp.float32)


def _fused_body(x_ref, a_ref, w1_ref, b1_ref, w2_ref, b2_ref,
                w3_ref, b3_ref, wc_ref, bc_ref, out_ref, aout_ref):
    f32 = jnp.float32
    bf16 = jnp.bfloat16
    c = out_ref.shape[1]
    k = a_ref.shape[1]

    _b = lambda r: r[...][:, None, None]
    w18 = jnp.pad(w1_ref[...], ((0, 1), (0, 0)))
    for s in range(out_ref.shape[0]):
        h = jnp.maximum(_cdot(w18.astype(bf16), a_ref[s].astype(bf16))
                        + _b(b1_ref), 0.0)
        h = jnp.maximum(_cdot(w2_ref[...].astype(bf16), h.astype(bf16))
                        + _b(b2_ref), 0.0)
        h = jnp.maximum(_cdot(w3_ref[...].astype(bf16), h.astype(bf16))
                        + _b(b3_ref), 0.0)
        aout3 = h * a_ref[s, k - 1:]
        aout_ref[s] = aout3

        xc3 = (_cdot_r(wc_ref[...].astype(bf16), x_ref[s].astype(bf16))
               + _b(bc_ref))
        xc3h = xc3.astype(bf16)
        aout3h = aout3.astype(bf16)
        for j in range(c):
            out_ref[s, j] = jax.lax.dot_general(
                aout3h[j], xc3h[j], (((0,), (0,)), ((), ())),
                preferred_element_type=f32)


def kernel(x, A, W1, b1, W2, b2, W3, b3, Wc, bc):
    n, c, t, v = x.shape
    k = A.shape[1]

    xt = jnp.swapaxes(x, 2, 3)
    w1t = jnp.swapaxes(W1, 0, 1)
    w2t = jnp.swapaxes(W2, 0, 1)
    w3t = jnp.swapaxes(W3, 0, 1)
    full = lambda a: pl.BlockSpec(a.shape, lambda i: (0,) * a.ndim)

    nb = 2
    outt, aout = pl.pallas_call(
        _fused_body,
        grid=(n // nb,),
        compiler_params=pltpu.CompilerParams(
            dimension_semantics=("parallel",)),
        in_specs=[
            pl.BlockSpec((nb, c, v, t), lambda i: (i, 0, 0, 0)),
            pl.BlockSpec((nb, k, v, v), lambda i: (i, 0, 0, 0)),
            full(w1t), full(b1), full(w2t), full(b2),
            full(w3t), full(b3), full(Wc), full(bc),
        ],
        out_specs=[
            pl.BlockSpec((nb, c, v, t), lambda i: (i, 0, 0, 0)),
            pl.BlockSpec((nb, c, v, v), lambda i: (i, 0, 0, 0)),
        ],
        out_shape=[
            jax.ShapeDtypeStruct((n, c, v, t), jnp.float32),
            jax.ShapeDtypeStruct((n, c, v, v), jnp.float32),
        ],
    )(xt, A, w1t, b1, w2t, b2, w3t, b3, Wc, bc)

    out = jnp.swapaxes(outt, 2, 3)
    return (out, aout)
